# baseline (speedup 1.0000x reference)
import functools

import jax
import jax.numpy as jnp
from jax import lax
from jax.experimental import pallas as pl
from jax.experimental.pallas import tpu as pltpu
from jax.experimental.pallas import tpu_sc as plsc

B, C, N = 8, 256, 4096
M = 1024
K = 4


def _fps_body(xs_ref, ys_ref, zs_ref, px_ref, py_ref, pz_ref, da_ref, db_ref):
    m_iota = lax.broadcasted_iota(jnp.int32, (B, M), 1)

    lx0 = xs_ref[:, 0:1]
    ly0 = ys_ref[:, 0:1]
    lz0 = zs_ref[:, 0:1]
    px_ref[...] = jnp.broadcast_to(lx0, (B, M))
    py_ref[...] = jnp.broadcast_to(ly0, (B, M))
    pz_ref[...] = jnp.broadcast_to(lz0, (B, M))
    da_ref[...] = jnp.full((B, N), 1e10, jnp.float32)

    lane = lax.broadcasted_iota(jnp.int32, (B, 128), 1).astype(jnp.float32)

    def merge(a, b):
        da, ia, xa, ya, za = a
        db, ib, xb, yb, zb = b
        take_a = (da > db) | ((da == db) & (ia < ib))
        return (jnp.where(take_a, da, db), jnp.where(take_a, ia, ib),
                jnp.where(take_a, xa, xb), jnp.where(take_a, ya, yb),
                jnp.where(take_a, za, zb))

    def step(src, dst, i, lx, ly, lz):
        accs = []
        for c in range(N // 128):
            sl = slice(c * 128, (c + 1) * 128)
            xc = xs_ref[:, sl]
            yc = ys_ref[:, sl]
            zc = zs_ref[:, sl]
            dc = ((xc - lx) ** 2 + (zc - lz) ** 2) + (yc - ly) ** 2
            dmin = jnp.minimum(src[:, sl], dc)
            dst[:, sl] = dmin
            cand = (dmin, lane + jnp.float32(128 * c), xc, yc, zc)
            if c < 4:
                accs.append(cand)
            else:
                accs[c % 4] = merge(accs[c % 4], cand)
        dv, iv, xv, yv, zv = merge(merge(accs[0], accs[1]),
                                   merge(accs[2], accs[3]))
        maxv = jnp.max(dv, axis=1, keepdims=True)
        key2 = jnp.where(dv == maxv, iv, jnp.float32(N))
        nxt = jnp.min(key2, axis=1, keepdims=True)
        m3 = key2 == nxt
        lx = jnp.sum(jnp.where(m3, xv, 0.0), axis=1, keepdims=True)
        ly = jnp.sum(jnp.where(m3, yv, 0.0), axis=1, keepdims=True)
        lz = jnp.sum(jnp.where(m3, zv, 0.0), axis=1, keepdims=True)

        rec = m_iota == i
        px_ref[...] = jnp.where(rec, lx, px_ref[...])
        py_ref[...] = jnp.where(rec, ly, py_ref[...])
        pz_ref[...] = jnp.where(rec, lz, pz_ref[...])
        return lx, ly, lz

    def body(t, carry):
        lx, ly, lz = carry
        i = 2 * t + 1
        lx, ly, lz = step(da_ref, db_ref, i, lx, ly, lz)
        lx, ly, lz = step(db_ref, da_ref, i + 1, lx, ly, lz)
        return lx, ly, lz

    carry = lax.fori_loop(0, (M - 2) // 2, body, (lx0, ly0, lz0))
    step(da_ref, db_ref, M - 1, *carry)


def _fps(xs, ys, zs, *, interpret=False):
    out = jax.ShapeDtypeStruct((B, M), jnp.float32)
    return pl.pallas_call(
        _fps_body,
        out_shape=(out, out, out),
        scratch_shapes=[pltpu.VMEM((B, N), jnp.float32),
                        pltpu.VMEM((B, N), jnp.float32)],
        interpret=interpret,
    )(xs, ys, zs)


_MT = 128


def _col(row):
    li = lax.broadcasted_iota(jnp.int32, (_MT, _MT), 1)
    si = lax.broadcasted_iota(jnp.int32, (_MT, _MT), 0)
    mat = jnp.where(li == si, jnp.broadcast_to(row, (_MT, _MT)), 0.0)
    return jnp.sum(mat, axis=1, keepdims=True)


def _bf16_trunc(x):
    bits = lax.bitcast_convert_type(x, jnp.int32)
    lsb = lax.shift_right_logical(bits, 16) & 1
    r = (bits + 32767 + lsb) & jnp.int32(-65536)
    return lax.bitcast_convert_type(r, jnp.float32)


def _knn_body(xs_ref, ys_ref, zs_ref, sx_ref, sy_ref, sz_ref, out_ref):
    xs = xs_ref[0]
    ys = ys_ref[0]
    zs = zs_ref[0]
    sx = _col(sx_ref[0, 0])
    sy = _col(sy_ref[0, 0])
    sz = _col(sz_ref[0, 0])
    n_iota = lax.broadcasted_iota(jnp.int32, (_MT, N), 1).astype(jnp.float32)

    xt, yt, zt = _bf16_trunc(xs), _bf16_trunc(ys), _bf16_trunc(zs)
    st_x, st_y, st_z = _bf16_trunc(sx), _bf16_trunc(sy), _bf16_trunc(sz)
    lhs = jnp.concatenate([st_x, st_y, st_z], axis=1).astype(jnp.bfloat16)
    rhs = jnp.concatenate([xt, yt, zt], axis=0).astype(jnp.bfloat16)
    inner = lax.dot_general(lhs, rhs, (((1,), (0,)), ((), ())),
                            preferred_element_type=jnp.float32)
    aa = sx * sx + sy * sy + sz * sz
    bb = xs * xs + ys * ys + zs * zs
    dist = -(aa - 2.0 * inner + bb)

    for j in range(K):
        maxv = jnp.max(dist, axis=1, keepdims=True)
        idxj = jnp.min(jnp.where(dist == maxv, n_iota, jnp.float32(N)),
                       axis=1, keepdims=True)
        out_ref[0, :, pl.ds(j, 1)] = idxj.astype(jnp.int32)
        if j + 1 < K:
            dist = jnp.where(n_iota == idxj, -jnp.inf, dist)


def _knn(xs, ys, zs, px, py, pz, *, interpret=False):
    grid = (B, M // _MT)
    pos_spec = pl.BlockSpec((1, 1, N), lambda b, mt: (b, 0, 0))
    s_spec = pl.BlockSpec((1, 1, 1, _MT), lambda b, mt: (b, mt, 0, 0))
    out_spec = pl.BlockSpec((1, _MT, K), lambda b, mt: (b, mt, 0))
    xs3 = xs.reshape(B, 1, N)
    ys3 = ys.reshape(B, 1, N)
    zs3 = zs.reshape(B, 1, N)
    px4 = px.reshape(B, M // _MT, 1, _MT)
    py4 = py.reshape(B, M // _MT, 1, _MT)
    pz4 = pz.reshape(B, M // _MT, 1, _MT)
    return pl.pallas_call(
        _knn_body,
        grid=grid,
        in_specs=[pos_spec] * 3 + [s_spec] * 3,
        out_specs=out_spec,
        out_shape=jax.ShapeDtypeStruct((B, M, K), jnp.int32),
        interpret=interpret,
    )(xs3, ys3, zs3, px4, py4, pz4)


_CB = 16
_TASKS = B * (C // _CB)
_NW = 32
_ROUNDS = _TASKS // _NW
_G = 16


def _pool_body(feats_hbm, idx_hbm, out_hbm, fblk, idxv, outv):
    wid = lax.axis_index("s") * 2 + lax.axis_index("c")

    for r in range(_ROUNDS):
        task = wid * _ROUNDS + r
        b = task // (C // _CB)
        cb = (task % (C // _CB)) * _CB
        pltpu.sync_copy(feats_hbm.at[b, pl.ds(cb, _CB)], fblk)
        pltpu.sync_copy(idx_hbm.at[b], idxv)

        l4 = lax.iota(jnp.int32, _G) * K

        def group(g, _):
            base = g * _G
            goff = g * (_G * K)
            i0 = plsc.load_gather(idxv, [l4 + goff])
            i1 = plsc.load_gather(idxv, [l4 + (goff + 1)])
            i2 = plsc.load_gather(idxv, [l4 + (goff + 2)])
            i3 = plsc.load_gather(idxv, [l4 + (goff + 3)])
            for ch in range(_CB):
                cv = jnp.full((_G,), ch, jnp.int32)
                v0 = plsc.load_gather(fblk, [cv, i0])
                v1 = plsc.load_gather(fblk, [cv, i1])
                v2 = plsc.load_gather(fblk, [cv, i2])
                v3 = plsc.load_gather(fblk, [cv, i3])
                outv[ch, pl.ds(base, _G)] = jnp.maximum(
                    jnp.maximum(v0, v1), jnp.maximum(v2, v3))
            return 0

        lax.fori_loop(0, M // _G, group, 0)
        pltpu.sync_copy(outv, out_hbm.at[b, pl.ds(cb, _CB)])


def _pool(feats, idx_flat):
    mesh = plsc.VectorSubcoreMesh(core_axis_name="c", subcore_axis_name="s")
    f = pl.kernel(
        _pool_body,
        out_type=jax.ShapeDtypeStruct((B, C, M), jnp.float32),
        mesh=mesh,
        scratch_types=[
            pltpu.VMEM((_CB, N), jnp.float32),
            pltpu.VMEM((M * K,), jnp.int32),
            pltpu.VMEM((_CB, M), jnp.float32),
        ],
        compiler_params=pltpu.CompilerParams(
            use_tc_tiling_on_sc=False, needs_layout_passes=False),
    )
    return f(feats, idx_flat)


def kernel(inputs, inputs_pos):
    xs = inputs_pos[:, 0, :]
    ys = inputs_pos[:, 1, :]
    zs = inputs_pos[:, 2, :]
    px, py, pz = _fps(xs, ys, zs)
    sampled_pos = jnp.stack([px, py, pz], axis=1)
    nn_idx = _knn(xs, ys, zs, px, py, pz)
    feat = _pool(inputs, nn_idx.reshape(B, M * K))
    return (feat, sampled_pos)


# speedup vs baseline: 1.0263x; 1.0263x over previous
import functools

import jax
import jax.numpy as jnp
from jax import lax
from jax.experimental import pallas as pl
from jax.experimental.pallas import tpu as pltpu
from jax.experimental.pallas import tpu_sc as plsc

B, C, N = 8, 256, 4096
M = 1024
K = 4


def _fps_body(xs_ref, ys_ref, zs_ref, px_ref, py_ref, pz_ref, da_ref, db_ref):
    m_iota = lax.broadcasted_iota(jnp.int32, (B, M), 1)

    lx0 = xs_ref[:, 0:1]
    ly0 = ys_ref[:, 0:1]
    lz0 = zs_ref[:, 0:1]
    px_ref[...] = jnp.broadcast_to(lx0, (B, M))
    py_ref[...] = jnp.broadcast_to(ly0, (B, M))
    pz_ref[...] = jnp.broadcast_to(lz0, (B, M))
    da_ref[...] = jnp.full((B, N), 1e10, jnp.float32)

    lane = lax.broadcasted_iota(jnp.int32, (B, 128), 1).astype(jnp.float32)

    def merge(a, b):
        da, ia, xa, ya, za = a
        db, ib, xb, yb, zb = b
        take_a = (da > db) | ((da == db) & (ia < ib))
        return (jnp.where(take_a, da, db), jnp.where(take_a, ia, ib),
                jnp.where(take_a, xa, xb), jnp.where(take_a, ya, yb),
                jnp.where(take_a, za, zb))

    def step(src, dst, i, lx, ly, lz):
        accs = []
        for c in range(N // 128):
            sl = slice(c * 128, (c + 1) * 128)
            xc = xs_ref[:, sl]
            yc = ys_ref[:, sl]
            zc = zs_ref[:, sl]
            dc = ((xc - lx) ** 2 + (zc - lz) ** 2) + (yc - ly) ** 2
            dmin = jnp.minimum(src[:, sl], dc)
            dst[:, sl] = dmin
            cand = (dmin, lane + jnp.float32(128 * c), xc, yc, zc)
            if c < 4:
                accs.append(cand)
            else:
                accs[c % 4] = merge(accs[c % 4], cand)
        dv, iv, xv, yv, zv = merge(merge(accs[0], accs[1]),
                                   merge(accs[2], accs[3]))
        maxv = jnp.max(dv, axis=1, keepdims=True)
        key2 = jnp.where(dv == maxv, iv, jnp.float32(N))
        nxt = jnp.min(key2, axis=1, keepdims=True)
        m3 = key2 == nxt
        lx = jnp.sum(jnp.where(m3, xv, 0.0), axis=1, keepdims=True)
        ly = jnp.sum(jnp.where(m3, yv, 0.0), axis=1, keepdims=True)
        lz = jnp.sum(jnp.where(m3, zv, 0.0), axis=1, keepdims=True)

        rec = m_iota == i
        px_ref[...] = jnp.where(rec, lx, px_ref[...])
        py_ref[...] = jnp.where(rec, ly, py_ref[...])
        pz_ref[...] = jnp.where(rec, lz, pz_ref[...])
        return lx, ly, lz

    def body(t, carry):
        lx, ly, lz = carry
        i = 2 * t + 1
        lx, ly, lz = step(da_ref, db_ref, i, lx, ly, lz)
        lx, ly, lz = step(db_ref, da_ref, i + 1, lx, ly, lz)
        return lx, ly, lz

    carry = lax.fori_loop(0, (M - 2) // 2, body, (lx0, ly0, lz0))
    step(da_ref, db_ref, M - 1, *carry)


def _fps(xs, ys, zs, *, interpret=False):
    out = jax.ShapeDtypeStruct((B, M), jnp.float32)
    return pl.pallas_call(
        _fps_body,
        out_shape=(out, out, out),
        scratch_shapes=[pltpu.VMEM((B, N), jnp.float32),
                        pltpu.VMEM((B, N), jnp.float32)],
        interpret=interpret,
    )(xs, ys, zs)


_MT = 128


def _col(row):
    li = lax.broadcasted_iota(jnp.int32, (_MT, _MT), 1)
    si = lax.broadcasted_iota(jnp.int32, (_MT, _MT), 0)
    mat = jnp.where(li == si, jnp.broadcast_to(row, (_MT, _MT)), 0.0)
    return jnp.sum(mat, axis=1, keepdims=True)


def _bf16_trunc(x):
    bits = lax.bitcast_convert_type(x, jnp.int32)
    lsb = lax.shift_right_logical(bits, 16) & 1
    r = (bits + 32767 + lsb) & jnp.int32(-65536)
    return lax.bitcast_convert_type(r, jnp.float32)


def _knn_body(xs_ref, ys_ref, zs_ref, sx_ref, sy_ref, sz_ref, out_ref):
    xs = xs_ref[0]
    ys = ys_ref[0]
    zs = zs_ref[0]
    sx = _col(sx_ref[0, 0])
    sy = _col(sy_ref[0, 0])
    sz = _col(sz_ref[0, 0])
    n_iota = lax.broadcasted_iota(jnp.int32, (_MT, N), 1).astype(jnp.float32)

    xt, yt, zt = _bf16_trunc(xs), _bf16_trunc(ys), _bf16_trunc(zs)
    st_x, st_y, st_z = _bf16_trunc(sx), _bf16_trunc(sy), _bf16_trunc(sz)
    lhs = jnp.concatenate([st_x, st_y, st_z], axis=1).astype(jnp.bfloat16)
    rhs = jnp.concatenate([xt, yt, zt], axis=0).astype(jnp.bfloat16)
    inner = lax.dot_general(lhs, rhs, (((1,), (0,)), ((), ())),
                            preferred_element_type=jnp.float32)
    aa = sx * sx + sy * sy + sz * sz
    bb = xs * xs + ys * ys + zs * zs
    dist = -(aa - 2.0 * inner + bb)

    for j in range(K):
        maxv = jnp.max(dist, axis=1, keepdims=True)
        idxj = jnp.min(jnp.where(dist == maxv, n_iota, jnp.float32(N)),
                       axis=1, keepdims=True)
        out_ref[0, :, pl.ds(j, 1)] = idxj.astype(jnp.int32)
        if j + 1 < K:
            dist = jnp.where(n_iota == idxj, -jnp.inf, dist)


def _knn(xs, ys, zs, px, py, pz, *, interpret=False):
    grid = (B, M // _MT)
    pos_spec = pl.BlockSpec((1, 1, N), lambda b, mt: (b, 0, 0))
    s_spec = pl.BlockSpec((1, 1, 1, _MT), lambda b, mt: (b, mt, 0, 0))
    out_spec = pl.BlockSpec((1, _MT, K), lambda b, mt: (b, mt, 0))
    xs3 = xs.reshape(B, 1, N)
    ys3 = ys.reshape(B, 1, N)
    zs3 = zs.reshape(B, 1, N)
    px4 = px.reshape(B, M // _MT, 1, _MT)
    py4 = py.reshape(B, M // _MT, 1, _MT)
    pz4 = pz.reshape(B, M // _MT, 1, _MT)
    return pl.pallas_call(
        _knn_body,
        grid=grid,
        in_specs=[pos_spec] * 3 + [s_spec] * 3,
        out_specs=out_spec,
        out_shape=jax.ShapeDtypeStruct((B, M, K), jnp.int32),
        interpret=interpret,
    )(xs3, ys3, zs3, px4, py4, pz4)


_CB = 8
_TASKS = B * (C // _CB)
_NW = 32
_ROUNDS = _TASKS // _NW
_G = 16


def _pool_body(feats_hbm, idx_hbm, out_hbm, fblk0, fblk1, idxv, outv,
               sem0, sem1):
    wid = lax.axis_index("s") * 2 + lax.axis_index("c")
    b = wid // (_NW // B)
    cb0 = (wid % (_NW // B)) * (_CB * _ROUNDS)

    pltpu.sync_copy(idx_hbm.at[b], idxv)
    fblks = (fblk0, fblk1)
    sems = (sem0, sem1)
    copies = [pltpu.async_copy(
        feats_hbm.at[b, pl.ds(cb0, _CB)], fblk0, sem0)]

    l4 = lax.iota(jnp.int32, _G) * K

    for r in range(_ROUNDS):
        cb = cb0 + r * _CB
        copies[r].wait()
        if r + 1 < _ROUNDS:
            copies.append(pltpu.async_copy(
                feats_hbm.at[b, pl.ds(cb + _CB, _CB)],
                fblks[(r + 1) % 2], sems[(r + 1) % 2]))
        fblk = fblks[r % 2]

        def group(g, _):
            base = g * _G
            goff = g * (_G * K)
            i0 = plsc.load_gather(idxv, [l4 + goff])
            i1 = plsc.load_gather(idxv, [l4 + (goff + 1)])
            i2 = plsc.load_gather(idxv, [l4 + (goff + 2)])
            i3 = plsc.load_gather(idxv, [l4 + (goff + 3)])
            for ch in range(_CB):
                cv = jnp.full((_G,), ch, jnp.int32)
                v0 = plsc.load_gather(fblk, [cv, i0])
                v1 = plsc.load_gather(fblk, [cv, i1])
                v2 = plsc.load_gather(fblk, [cv, i2])
                v3 = plsc.load_gather(fblk, [cv, i3])
                outv[ch, pl.ds(base, _G)] = jnp.maximum(
                    jnp.maximum(v0, v1), jnp.maximum(v2, v3))
            return 0

        lax.fori_loop(0, M // _G, group, 0)
        pltpu.sync_copy(outv, out_hbm.at[b, pl.ds(cb, _CB)])


def _pool(feats, idx_flat):
    mesh = plsc.VectorSubcoreMesh(core_axis_name="c", subcore_axis_name="s")
    f = pl.kernel(
        _pool_body,
        out_type=jax.ShapeDtypeStruct((B, C, M), jnp.float32),
        mesh=mesh,
        scratch_types=[
            pltpu.VMEM((_CB, N), jnp.float32),
            pltpu.VMEM((_CB, N), jnp.float32),
            pltpu.VMEM((M * K,), jnp.int32),
            pltpu.VMEM((_CB, M), jnp.float32),
            pltpu.SemaphoreType.DMA,
            pltpu.SemaphoreType.DMA,
        ],
        compiler_params=pltpu.CompilerParams(
            use_tc_tiling_on_sc=False, needs_layout_passes=False),
    )
    return f(feats, idx_flat)


def kernel(inputs, inputs_pos):
    xs = inputs_pos[:, 0, :]
    ys = inputs_pos[:, 1, :]
    zs = inputs_pos[:, 2, :]
    px, py, pz = _fps(xs, ys, zs)
    sampled_pos = jnp.stack([px, py, pz], axis=1)
    nn_idx = _knn(xs, ys, zs, px, py, pz)
    feat = _pool(inputs, nn_idx.reshape(B, M * K))
    return (feat, sampled_pos)
